# baseline (speedup 1.0000x reference)
import functools

import jax
import jax.numpy as jnp
from jax import lax
from jax.experimental import pallas as pl
from jax.experimental.pallas import tpu as pltpu
from jax.experimental.pallas import tpu_sc as plsc

_R = 4
_NB = 2
_SPLITS = ((0, 128), (128, 72))


def _emb_call(rows_pw, idx, table):
    B0, S = idx.shape
    V, D = table.shape
    mesh = plsc.VectorSubcoreMesh(core_axis_name="c", subcore_axis_name="s")
    NC = 2
    n_chunks = rows_pw // _R
    n_outer = n_chunks // _NB
    DP = 2 * D

    @functools.partial(
        pl.kernel,
        out_type=jax.ShapeDtypeStruct((B0, S, DP), jnp.float32),
        mesh=mesh,
        scratch_types=[
            pltpu.VMEM((rows_pw, S), jnp.int32),
            pltpu.VMEM((_NB, _R, S, D), jnp.float32),
            [pltpu.SemaphoreType.DMA] * _NB,
            [pltpu.SemaphoreType.DMA] * _NB,
        ],
        compiler_params=pltpu.CompilerParams(use_tc_tiling_on_sc=False),
    )
    def emb(idx_hbm, table_hbm, out_hbm, idx_v, rows_v, gsem, ssem):
        wid = lax.axis_index("s") * NC + lax.axis_index("c")
        b00 = wid * rows_pw

        def fire(cc, buf):
            for i in range(_R):
                for (o, w) in _SPLITS:
                    pltpu.async_copy(
                        table_hbm.at[idx_v.at[cc * _R + i, pl.ds(o, w)]],
                        rows_v.at[buf, i, pl.ds(o, w)],
                        gsem[buf],
                    )

        def wait_gathers(buf):
            for i in range(_R):
                for (o, w) in _SPLITS:
                    pltpu.make_async_copy(
                        table_hbm.at[idx_v.at[i, pl.ds(o, w)]],
                        rows_v.at[buf, i, pl.ds(o, w)],
                        gsem[buf],
                    ).wait()

        def fire_store(cc, buf):
            b0 = b00 + cc * _R
            pltpu.async_copy(
                rows_v.at[buf],
                out_hbm.at[pl.ds(b0, _R), :, pl.ds(0, D)],
                ssem[buf],
            )

        def wait_store(buf):
            pltpu.make_async_copy(
                rows_v.at[buf],
                out_hbm.at[pl.ds(0, _R), :, pl.ds(0, D)],
                ssem[buf],
            ).wait()

        pltpu.sync_copy(idx_hbm.at[pl.ds(b00, rows_pw)], idx_v)
        for b in range(_NB - 1):
            fire(b, b)

        def body(it, _):
            for u in range(_NB):
                cc = _NB * it + u
                nc = cc + (_NB - 1)
                nbuf = (u + _NB - 1) % _NB
                wait_gathers(u)
                fire_store(cc, u)

                @pl.when(nc < n_chunks)
                def _():
                    @pl.when(cc >= 1)
                    def _():
                        wait_store(nbuf)

                    fire(nc, nbuf)

            return 0

        lax.fori_loop(0, n_outer, body, 0)
        for b in range(_NB):
            wait_store(b)

    return emb(idx, table)


def kernel(input_ids, table):
    B0, S = input_ids.shape
    NW = 32
    rows_pw = B0 // NW
    assert rows_pw % (_NB * _R) == 0
    out_p = _emb_call(rows_pw, input_ids, table)
    return out_p[..., : table.shape[1]]


# speedup vs baseline: 1.0007x; 1.0007x over previous
import functools

import jax
import jax.numpy as jnp
from jax import lax
from jax.experimental import pallas as pl
from jax.experimental.pallas import tpu as pltpu
from jax.experimental.pallas import tpu_sc as plsc

_R = 2
_NB = 4
_SPLITS = ((0, 128), (128, 72))


def _emb_call(rows_pw, idx, table):
    B0, S = idx.shape
    V, D = table.shape
    mesh = plsc.VectorSubcoreMesh(core_axis_name="c", subcore_axis_name="s")
    NC = 2
    n_chunks = rows_pw // _R
    n_outer = n_chunks // _NB
    DP = 2 * D

    @functools.partial(
        pl.kernel,
        out_type=jax.ShapeDtypeStruct((B0, S, DP), jnp.float32),
        mesh=mesh,
        scratch_types=[
            pltpu.VMEM((rows_pw, S), jnp.int32),
            pltpu.VMEM((_NB, _R, S, D), jnp.float32),
            [pltpu.SemaphoreType.DMA] * _NB,
            [pltpu.SemaphoreType.DMA] * _NB,
        ],
        compiler_params=pltpu.CompilerParams(use_tc_tiling_on_sc=False),
    )
    def emb(idx_hbm, table_hbm, out_hbm, idx_v, rows_v, gsem, ssem):
        wid = lax.axis_index("s") * NC + lax.axis_index("c")
        b00 = wid * rows_pw

        def fire(cc, buf):
            for i in range(_R):
                for (o, w) in _SPLITS:
                    pltpu.async_copy(
                        table_hbm.at[idx_v.at[cc * _R + i, pl.ds(o, w)]],
                        rows_v.at[buf, i, pl.ds(o, w)],
                        gsem[buf],
                    )

        def wait_gathers(buf):
            for i in range(_R):
                for (o, w) in _SPLITS:
                    pltpu.make_async_copy(
                        table_hbm.at[idx_v.at[i, pl.ds(o, w)]],
                        rows_v.at[buf, i, pl.ds(o, w)],
                        gsem[buf],
                    ).wait()

        def fire_store(cc, buf):
            b0 = b00 + cc * _R
            pltpu.async_copy(
                rows_v.at[buf],
                out_hbm.at[pl.ds(b0, _R), :, pl.ds(0, D)],
                ssem[buf],
            )

        def wait_store(buf):
            pltpu.make_async_copy(
                rows_v.at[buf],
                out_hbm.at[pl.ds(0, _R), :, pl.ds(0, D)],
                ssem[buf],
            ).wait()

        pltpu.sync_copy(idx_hbm.at[pl.ds(b00, rows_pw)], idx_v)
        for b in range(_NB - 1):
            fire(b, b)

        def body(it, _):
            for u in range(_NB):
                cc = _NB * it + u
                nc = cc + (_NB - 1)
                nbuf = (u + _NB - 1) % _NB
                wait_gathers(u)
                fire_store(cc, u)

                @pl.when(nc < n_chunks)
                def _():
                    @pl.when(cc >= 1)
                    def _():
                        wait_store(nbuf)

                    fire(nc, nbuf)

            return 0

        lax.fori_loop(0, n_outer, body, 0)
        for b in range(_NB):
            wait_store(b)

    return emb(idx, table)


def kernel(input_ids, table):
    B0, S = input_ids.shape
    NW = 32
    rows_pw = B0 // NW
    assert rows_pw % (_NB * _R) == 0
    out_p = _emb_call(rows_pw, input_ids, table)
    return out_p[..., : table.shape[1]]


# speedup vs baseline: 1.0018x; 1.0011x over previous
import functools

import jax
import jax.numpy as jnp
from jax import lax
from jax.experimental import pallas as pl
from jax.experimental.pallas import tpu as pltpu
from jax.experimental.pallas import tpu_sc as plsc

_R = 2
_NB = 4
_SPLITS = ((0, 128), (128, 72))


def _emb_call(rows_pw, idx, table):
    B0, S = idx.shape
    V, D = table.shape
    mesh = plsc.VectorSubcoreMesh(core_axis_name="c", subcore_axis_name="s")
    NC = 2
    n_chunks = rows_pw // _R
    DP = 2 * D

    @functools.partial(
        pl.kernel,
        out_type=jax.ShapeDtypeStruct((B0, S, DP), jnp.float32),
        mesh=mesh,
        scratch_types=[
            pltpu.VMEM((rows_pw, S), jnp.int32),
            pltpu.VMEM((_NB, _R, S, D), jnp.float32),
            pltpu.SemaphoreType.DMA((_NB,)),
            pltpu.SemaphoreType.DMA((_NB,)),
        ],
        compiler_params=pltpu.CompilerParams(use_tc_tiling_on_sc=False),
    )
    def emb(idx_hbm, table_hbm, out_hbm, idx_v, rows_v, gsem, ssem):
        wid = lax.axis_index("s") * NC + lax.axis_index("c")
        b00 = wid * rows_pw

        def fire(cc, buf):
            for i in range(_R):
                for (o, w) in _SPLITS:
                    pltpu.async_copy(
                        table_hbm.at[idx_v.at[cc * _R + i, pl.ds(o, w)]],
                        rows_v.at[buf, i, pl.ds(o, w)],
                        gsem.at[buf],
                    )

        def wait_gathers(buf):
            for i in range(_R):
                for (o, w) in _SPLITS:
                    pltpu.make_async_copy(
                        table_hbm.at[idx_v.at[i, pl.ds(o, w)]],
                        rows_v.at[buf, i, pl.ds(o, w)],
                        gsem.at[buf],
                    ).wait()

        def fire_store(cc, buf):
            b0 = b00 + cc * _R
            pltpu.async_copy(
                rows_v.at[buf],
                out_hbm.at[pl.ds(b0, _R), :, pl.ds(0, D)],
                ssem.at[buf],
            )

        def wait_store(buf):
            pltpu.make_async_copy(
                rows_v.at[buf],
                out_hbm.at[pl.ds(0, _R), :, pl.ds(0, D)],
                ssem.at[buf],
            ).wait()

        pltpu.sync_copy(idx_hbm.at[pl.ds(b00, rows_pw)], idx_v)
        for b in range(_NB - 1):
            fire(b, b)

        def body(cc, _):
            buf = lax.rem(cc, _NB)
            nc = cc + (_NB - 1)
            nbuf = lax.rem(nc, _NB)
            wait_gathers(buf)
            fire_store(cc, buf)

            @pl.when(nc < n_chunks)
            def _():
                @pl.when(cc >= 1)
                def _():
                    wait_store(nbuf)

                fire(nc, nbuf)

            return 0

        lax.fori_loop(0, n_chunks, body, 0)
        for b in range(_NB):
            wait_store(b)

    return emb(idx, table)


def kernel(input_ids, table):
    B0, S = input_ids.shape
    NW = 32
    rows_pw = B0 // NW
    assert rows_pw % (_NB * _R) == 0
    out_p = _emb_call(rows_pw, input_ids, table)
    return out_p[..., : table.shape[1]]
